# Initial kernel scaffold; baseline (speedup 1.0000x reference)
import jax
import jax.numpy as jnp
from jax.experimental import pallas as pl


def kernel(text, table, W, b):
    raise NotImplementedError("write your pallas kernel here")


# speedup vs baseline: 1.5250x; 1.5250x over previous
import functools

import jax
import jax.numpy as jnp
from jax import lax
from jax.experimental import pallas as pl
from jax.experimental.pallas import tpu as pltpu
from jax.experimental.pallas import tpu_sc as plsc

NC = 2
NS = 16
L = 16
NW = NC * NS


@functools.partial(jax.jit, static_argnames=())
def kernel(text, table, W, b):
    S, B = text.shape
    V, D = table.shape
    O = W.shape[0]
    assert B % NW == 0 and D == 2 * L
    bpw = B // NW

    mesh = plsc.VectorSubcoreMesh(
        core_axis_name="c", subcore_axis_name="s",
        num_cores=NC, num_subcores=NS)

    @functools.partial(
        pl.kernel,
        out_type=jax.ShapeDtypeStruct((B * O,), jnp.float32),
        mesh=mesh,
        compiler_params=pltpu.CompilerParams(
            needs_layout_passes=False, use_tc_tiling_on_sc=False),
        scratch_types=[
            pltpu.VMEM((S, bpw), jnp.int32),
            pltpu.VMEM((bpw, D), jnp.float32),
            pltpu.VMEM((bpw, D), jnp.float32),
            pltpu.VMEM((O, D), jnp.float32),
            pltpu.VMEM((L,), jnp.float32),
            pltpu.VMEM((bpw * D,), jnp.float32),
            pltpu.VMEM((bpw * O,), jnp.float32),
        ],
    )
    def fasttext_sc(text_h, table_h, w_h, b_h, out_h,
                    idx_v, rows_v, acc_v, w_v, b_v, flat_v, out_v):
        wid = lax.axis_index("s") * NC + lax.axis_index("c")
        base = wid * bpw

        pltpu.sync_copy(text_h.at[:, pl.ds(base, bpw)], idx_v)
        pltpu.sync_copy(w_h, w_v)
        pltpu.sync_copy(b_h, b_v.at[pl.ds(0, O)])

        pltpu.sync_copy(table_h.at[idx_v.at[0]], acc_v)

        unroll = 8
        n_out = bpw // unroll

        def seq_step(s, _):
            pltpu.sync_copy(table_h.at[idx_v.at[s]], rows_v)

            def acc_rows(i, _):
                r0 = i * unroll
                for k in range(unroll):
                    for h in range(D // L):
                        plsc.addupdate(
                            acc_v.at[r0 + k, pl.ds(h * L, L)],
                            rows_v[r0 + k, pl.ds(h * L, L)])
                return 0

            lax.fori_loop(0, n_out, acc_rows, 0, unroll=1)
            return 0

        lax.fori_loop(1, S, seq_step, 0, unroll=1)

        def flat_rows(i, _):
            r0 = i * unroll
            for k in range(unroll):
                for h in range(D // L):
                    flat_v[pl.ds((r0 + k) * D + h * L, L)] = (
                        acc_v[r0 + k, pl.ds(h * L, L)])
            return 0

        lax.fori_loop(0, n_out, flat_rows, 0, unroll=1)

        inv_s = jnp.float32(1.0 / S)
        lanes = lax.iota(jnp.int32, L)
        w_rows = [[w_v[o, pl.ds(h * L, L)] for h in range(D // L)]
                  for o in range(O)]
        ws = [[w_rows[o][d // L][d % L] for d in range(D)] for o in range(O)]
        b_vec = b_v[pl.ds(0, L)]
        bs = [b_vec[o] for o in range(O)]
        for g in range(bpw // L):
            row_idx = (g * L + lanes) * D
            outs = [jnp.zeros((L,), jnp.float32) for _ in range(O)]
            for d in range(D):
                vals = plsc.load_gather(flat_v, [row_idx + d])
                for o in range(O):
                    outs[o] = outs[o] + vals * ws[o][d]
            for o in range(O):
                res = outs[o] * inv_s + bs[o]
                plsc.store_scatter(out_v, [(g * L + lanes) * O + o], res)

        pltpu.sync_copy(out_v, out_h.at[pl.ds(base * O, bpw * O)])

    return fasttext_sc(text, table, W, b).reshape(B, O)
